# Initial kernel scaffold; baseline (speedup 1.0000x reference)
import jax
import jax.numpy as jnp
from jax.experimental import pallas as pl


def kernel(x, out_table, in_table, pe):
    raise NotImplementedError("write your pallas kernel here")


# speedup vs baseline: 1.2586x; 1.2586x over previous
import functools

import jax
import jax.numpy as jnp
from jax import lax
from jax.experimental import pallas as pl
from jax.experimental.pallas import tpu as pltpu
from jax.experimental.pallas import tpu_sc as plsc

HID = 128
K = 128
NC = 2
NS = 16
NW = NC * NS


def _encoder_call(n_pad, cpw):
    mesh = plsc.VectorSubcoreMesh(core_axis_name="c", subcore_axis_name="s")

    @functools.partial(
        pl.kernel,
        mesh=mesh,
        out_type=jax.ShapeDtypeStruct((n_pad, HID), jnp.float32),
        scratch_types=[
            pltpu.VMEM((3, K), jnp.int32),
            pltpu.VMEM((K, HID), jnp.float32),
            pltpu.VMEM((K, HID), jnp.float32),
            pltpu.VMEM((K, HID), jnp.float32),
            pltpu.SemaphoreType.DMA,
            pltpu.SemaphoreType.DMA,
            pltpu.SemaphoreType.DMA,
        ],
    )
    def enc(idx_hbm, pe_hbm, ot_hbm, it_hbm, out_hbm,
            idx_v, buf_a, buf_b, buf_c, sem_a, sem_b, sem_c):
        wid = lax.axis_index("s") * NC + lax.axis_index("c")

        def chunk_body(c, carry):
            t = wid * cpw + c
            pltpu.sync_copy(idx_hbm.at[t], idx_v)
            cp_a = pltpu.async_copy(pe_hbm.at[idx_v.at[0]], buf_a, sem_a)
            cp_b = pltpu.async_copy(ot_hbm.at[idx_v.at[1]], buf_b, sem_b)
            cp_c = pltpu.async_copy(it_hbm.at[idx_v.at[2]], buf_c, sem_c)
            cp_a.wait()
            cp_b.wait()
            cp_c.wait()

            def add_body(j, carry2):
                for l in range(HID // 16):
                    s = pl.ds(l * 16, 16)
                    plsc.addupdate(buf_a.at[j, s], buf_b[j, s] + buf_c[j, s])
                return carry2

            lax.fori_loop(0, K, add_body, 0, unroll=False)
            pltpu.sync_copy(buf_a, out_hbm.at[pl.ds(t * K, K)])
            return carry

        lax.fori_loop(0, cpw, chunk_body, 0, unroll=False)

    return enc


def kernel(x, out_table, in_table, pe):
    n = x.shape[0]
    block = NW * K
    n_pad = ((n + block - 1) // block) * block
    cpw = n_pad // block

    idx = x.astype(jnp.int32).T
    idx = jnp.pad(idx, ((0, 0), (0, n_pad - n)))
    idx = idx.reshape(3, n_pad // K, K).transpose(1, 0, 2)

    out = _encoder_call(n_pad, cpw)(idx, pe, out_table, in_table)
    return out[:n]
